# baseline (speedup 1.0000x reference)
import jax
import jax.numpy as jnp
import numpy as np
from jax.experimental import pallas as pl
from jax.experimental.pallas import tpu as pltpu

N_NODES = 16
FANOUT = 4
EPSILON = 0.01
USAGE_BETA = 0.001
D_MODEL = 1024
EMB_DIMS = 128
BATCH_TOKENS = 2048

BB = 1024


def _col(plane, hot_row):
    return jnp.sum(plane * hot_row, axis=1, keepdims=True)


def _dispatch_math(n, proj, src, ad_prev, embsT_ref, wrow_ref, negb_ref,
                   hot_ref, ad_ref, scaled):
    BBl = proj.shape[0]
    iota = jax.lax.broadcasted_iota(jnp.int32, (BBl, N_NODES), 1)
    logits = jnp.dot(proj, embsT_ref[...], preferred_element_type=jnp.float32)
    lm = logits + negb_ref[pl.ds(n, 1), :]
    mx = jnp.max(lm, axis=1, keepdims=True)
    e = jnp.exp(lm - mx)
    ecols = [_col(e, hot_ref[pl.ds(n + dj, 1), :]) for dj in range(1, 5)]
    s = (ecols[0] + ecols[1]) + (ecols[2] + ecols[3])
    attn = e / s
    aw = attn * wrow_ref[pl.ds(n, 1), :]
    awcols = [_col(aw, hot_ref[pl.ds(n + dj, 1), :]) for dj in range(1, 5)]
    saw = (awcols[0] + awcols[1]) + (awcols[2] + awcols[3])
    attn2 = aw / (saw + 1e-9)
    if scaled:
        asent = attn2 * src
    else:
        asent = attn2

    ad_pre = ad_prev + asent
    m1 = jnp.max(ad_pre, axis=1, keepdims=True)
    i1 = jnp.min(jnp.where(ad_pre == m1, iota, N_NODES), axis=1, keepdims=True)
    hot1 = iota == i1
    ad2 = jnp.where(hot1, -1.0, ad_pre)
    m2 = jnp.max(ad2, axis=1, keepdims=True)
    i2 = jnp.min(jnp.where(ad2 == m2, iota, N_NODES), axis=1, keepdims=True)
    hot2 = iota == i2
    maskf = jnp.where((hot1 | hot2) & (ad_pre > EPSILON), 1.0, 0.0)
    kept = ad_pre * maskf
    scale = 1.0 / (jnp.sum(kept, axis=1, keepdims=True) + 1e-9)
    ad_ref[...] = (ad_pre * maskf) * scale
    arecv = (asent * maskf) * scale
    return asent, arecv, maskf, scale


def _stage_kernel(x_ref, wa1_ref, wa2_ref, wt_ref, bt_ref, embsT_ref,
                  wrow_ref, negb_ref, zrow_ref, hot_ref, out_ref, agg0, agg1,
                  agg2, agg3, aga_ref, ad_ref):
    n = pl.program_id(1)
    aggs = (agg0, agg1, agg2, agg3)

    @pl.when(n == 0)
    def _():
        xb = x_ref[...]
        proj = jnp.dot(xb, wa1_ref[...], preferred_element_type=jnp.float32)
        zeros = jnp.zeros((xb.shape[0], N_NODES), jnp.float32)
        asent, arecv, maskf, scale = _dispatch_math(
            0, proj, None, zeros, embsT_ref, wrow_ref, negb_ref, hot_ref,
            ad_ref, scaled=False)
        aga_ref[...] = arecv
        for t in range(1, FANOUT + 1):
            hr = hot_ref[t:t + 1, :]
            a_t = _col(arecv, hr)
            aggs[t % 4][...] = a_t * xb

    @pl.when(n > 0)
    def _():
        hot_n = hot_ref[pl.ds(n, 1), :]
        src = _col(aga_ref[...], hot_n)
        slot_n = jax.lax.rem(n, 4)
        for j in range(4):
            @pl.when(slot_n == j)
            def _(j=j):
                aggr = aggs[j][...] / (src + 1e-9)
                out = jnp.dot(aggr, wt_ref[0],
                              preferred_element_type=jnp.float32)
                out = jnp.maximum(out + bt_ref[pl.ds(n, 1), :], 0.0)
                proj = jnp.dot(out, wa2_ref[...],
                               preferred_element_type=jnp.float32)
                ad_prev = ad_ref[...] * zrow_ref[pl.ds(n, 1), :]
                asent, arecv, maskf, scale = _dispatch_math(
                    n, proj, src, ad_prev, embsT_ref, wrow_ref, negb_ref,
                    hot_ref, ad_ref, scaled=True)
                nf = maskf * scale
                aga_ref[...] = (aga_ref[...] + asent) * nf
                for dj in range(1, 4):
                    @pl.when(n + dj < N_NODES)
                    def _(dj=dj, j=j):
                        t_slot = (j + dj) % 4
                        hr_t = hot_ref[pl.ds(n + dj, 1), :]
                        a_t = _col(asent, hr_t)
                        nf_t = _col(nf, hr_t)
                        aggs[t_slot][...] = (aggs[t_slot][...]
                                             + a_t * out) * nf_t
                @pl.when(n + 4 < N_NODES)
                def _(j=j):
                    hr_t = hot_ref[pl.ds(n + 4, 1), :]
                    a_t = _col(asent, hr_t)
                    nf_t = _col(nf, hr_t)
                    aggs[j][...] = (a_t * out) * nf_t

        @pl.when(n == N_NODES - 2)
        def _():
            hr = hot_ref[N_NODES - 1:N_NODES, :]
            den = _col(aga_ref[...], hr)
            out_ref[...] = aggs[(N_NODES - 1) % 4][...] / (den + 1e-9)


@jax.jit
def kernel(x, node_embs, W_attn1, W_attn2, W_trans, b_trans, node_usages):
    B = x.shape[0]
    embsT = node_embs.T
    rows = []
    for nid in range(N_NODES - 1):
        t0, t1 = nid + 1, min(nid + 1 + FANOUT, N_NODES)
        usages = node_usages[t0:t1]
        w = 1.0 / (usages + USAGE_BETA)
        w = w / jnp.sum(w)
        rows.append(jnp.zeros((N_NODES,), jnp.float32).at[t0:t1].set(w))
    rows.append(jnp.zeros((N_NODES,), jnp.float32))
    wrow = jnp.stack(rows)

    tmask_np = np.zeros((N_NODES, N_NODES), np.float32)
    for nid in range(N_NODES - 1):
        tmask_np[nid, nid + 1:min(nid + 1 + FANOUT, N_NODES)] = 1.0
    negb = jnp.asarray((1.0 - tmask_np) * -1e30)
    zrow = jnp.asarray(1.0 - np.eye(N_NODES, dtype=np.float32))
    hot_np = np.zeros((2 * N_NODES, N_NODES), np.float32)
    hot_np[:N_NODES] = np.eye(N_NODES, dtype=np.float32)
    hot = jnp.asarray(hot_np)

    nb = B // BB
    grid = (nb, N_NODES - 1)
    out = pl.pallas_call(
        _stage_kernel,
        grid=grid,
        in_specs=[
            pl.BlockSpec((BB, D_MODEL), lambda b, n: (b, 0)),
            pl.BlockSpec((D_MODEL, EMB_DIMS), lambda b, n: (0, 0)),
            pl.BlockSpec((D_MODEL, EMB_DIMS), lambda b, n: (0, 0)),
            pl.BlockSpec((1, D_MODEL, D_MODEL), lambda b, n: (n, 0, 0)),
            pl.BlockSpec((N_NODES, D_MODEL), lambda b, n: (0, 0)),
            pl.BlockSpec((EMB_DIMS, N_NODES), lambda b, n: (0, 0)),
            pl.BlockSpec((N_NODES, N_NODES), lambda b, n: (0, 0)),
            pl.BlockSpec((N_NODES, N_NODES), lambda b, n: (0, 0)),
            pl.BlockSpec((N_NODES, N_NODES), lambda b, n: (0, 0)),
            pl.BlockSpec((2 * N_NODES, N_NODES), lambda b, n: (0, 0)),
        ],
        out_specs=pl.BlockSpec((BB, D_MODEL), lambda b, n: (b, 0)),
        out_shape=jax.ShapeDtypeStruct((B, D_MODEL), jnp.float32),
        scratch_shapes=[
            pltpu.VMEM((BB, D_MODEL), jnp.float32),
            pltpu.VMEM((BB, D_MODEL), jnp.float32),
            pltpu.VMEM((BB, D_MODEL), jnp.float32),
            pltpu.VMEM((BB, D_MODEL), jnp.float32),
            pltpu.VMEM((BB, N_NODES), jnp.float32),
            pltpu.VMEM((BB, N_NODES), jnp.float32),
        ],
        compiler_params=pltpu.CompilerParams(
            dimension_semantics=("arbitrary", "arbitrary"),
        ),
    )(x, W_attn1, W_attn2, W_trans, b_trans, embsT, wrow, negb, zrow, hot)
    return out


# speedup vs baseline: 4.0153x; 4.0153x over previous
import jax
import jax.numpy as jnp
import numpy as np
from jax.experimental import pallas as pl
from jax.experimental.pallas import tpu as pltpu

N_NODES = 16
FANOUT = 4
EPSILON = 0.01
USAGE_BETA = 0.001
D_MODEL = 1024
EMB_DIMS = 128
BATCH_TOKENS = 2048

BB = 1024


def _col(plane, hot_row):
    return jnp.sum(plane * hot_row, axis=1, keepdims=True)


def _dispatch_math(n, proj, denom, ad_prev, embsT_ref, wrow_ref, negb_ref,
                   hot_ref, ad_ref):
    BBl = proj.shape[0]
    iota = jax.lax.broadcasted_iota(jnp.int32, (BBl, N_NODES), 1)
    logits = jnp.dot(proj, embsT_ref[...], preferred_element_type=jnp.float32)
    lm = logits + negb_ref[pl.ds(n, 1), :]
    mx = jnp.max(lm, axis=1, keepdims=True)
    e = jnp.exp(lm - mx)
    ecols = [_col(e, hot_ref[pl.ds(n + dj, 1), :]) for dj in range(1, 5)]
    s = (ecols[0] + ecols[1]) + (ecols[2] + ecols[3])
    attn = e / s
    aw = attn * wrow_ref[pl.ds(n, 1), :]
    awcols = [_col(aw, hot_ref[pl.ds(n + dj, 1), :]) for dj in range(1, 5)]
    saw = (awcols[0] + awcols[1]) + (awcols[2] + awcols[3])
    attn2 = aw / (saw + 1e-9)
    asent = attn2 * denom

    ad_pre = ad_prev + asent
    m1 = jnp.max(ad_pre, axis=1, keepdims=True)
    i1 = jnp.min(jnp.where(ad_pre == m1, iota, N_NODES), axis=1, keepdims=True)
    hot1 = iota == i1
    ad2 = jnp.where(hot1, -1.0, ad_pre)
    m2 = jnp.max(ad2, axis=1, keepdims=True)
    i2 = jnp.min(jnp.where(ad2 == m2, iota, N_NODES), axis=1, keepdims=True)
    hot2 = iota == i2
    maskf = jnp.where((hot1 | hot2) & (ad_pre > EPSILON), 1.0, 0.0)
    kept = ad_pre * maskf
    scale = 1.0 / (jnp.sum(kept, axis=1, keepdims=True) + 1e-9)
    nf = maskf * scale
    ad_ref[...] = (ad_pre * maskf) * scale
    return asent, nf


def _stage_kernel(x_ref, wa1_ref, wa2_ref, wt_ref, bt_ref, embsT_ref,
                  wrow_ref, negb_ref, zrow_ref, hot_ref, out_ref, ring0,
                  ring1, ring2, ring3, coef0, coef1, coef2, coef3, ad_ref):
    n = pl.program_id(1)
    rings = (ring0, ring1, ring2, ring3)
    coefs = (coef0, coef1, coef2, coef3)

    @pl.when(n == 0)
    def _():
        xb = x_ref[...]
        ring0[...] = xb
        ring1[...] = jnp.zeros_like(xb)
        ring2[...] = jnp.zeros_like(xb)
        ring3[...] = jnp.zeros_like(xb)
        proj = jnp.dot(xb, wa1_ref[...], preferred_element_type=jnp.float32)
        ones = jnp.ones((xb.shape[0], 1), jnp.float32)
        zeros = jnp.zeros((xb.shape[0], N_NODES), jnp.float32)
        asent, nf = _dispatch_math(0, proj, ones, zeros, embsT_ref, wrow_ref,
                                   negb_ref, hot_ref, ad_ref)
        coef0[...] = asent * nf
        coef1[...] = jnp.zeros_like(asent)
        coef2[...] = jnp.zeros_like(asent)
        coef3[...] = jnp.zeros_like(asent)

    @pl.when(n > 0)
    def _():
        hot_n = hot_ref[pl.ds(n, 1), :]
        cj = [_col(coefs[j][...], hot_n) for j in range(4)]
        denom = (cj[0] + cj[1]) + (cj[2] + cj[3])
        aggr = (cj[0] * ring0[...] + cj[1] * ring1[...]
                + cj[2] * ring2[...] + cj[3] * ring3[...])
        aggr = aggr / (denom + 1e-9)
        out = jnp.dot(aggr, wt_ref[0], preferred_element_type=jnp.float32)
        out = jnp.maximum(out + bt_ref[pl.ds(n, 1), :], 0.0)
        proj = jnp.dot(out, wa2_ref[...], preferred_element_type=jnp.float32)
        ad_prev = ad_ref[...] * zrow_ref[pl.ds(n, 1), :]
        asent, nf = _dispatch_math(n, proj, denom, ad_prev, embsT_ref,
                                   wrow_ref, negb_ref, hot_ref, ad_ref)
        slot = jax.lax.rem(n, 4)
        for j in range(4):
            @pl.when(slot == j)
            def _(j=j):
                rings[j][...] = out
                coefs[j][...] = asent * nf

            @pl.when(slot != j)
            def _(j=j):
                coefs[j][...] = coefs[j][...] * nf

        @pl.when(n == N_NODES - 2)
        def _():
            c15 = [coefs[j][:, N_NODES - 1:N_NODES] for j in range(4)]
            den = (c15[0] + c15[1]) + (c15[2] + c15[3])
            outf = (c15[0] * ring0[...] + c15[1] * ring1[...]
                    + c15[2] * ring2[...] + c15[3] * ring3[...])
            out_ref[...] = outf / (den + 1e-9)


@jax.jit
def kernel(x, node_embs, W_attn1, W_attn2, W_trans, b_trans, node_usages):
    B = x.shape[0]
    embsT = node_embs.T
    rows = []
    for nid in range(N_NODES - 1):
        t0, t1 = nid + 1, min(nid + 1 + FANOUT, N_NODES)
        usages = node_usages[t0:t1]
        w = 1.0 / (usages + USAGE_BETA)
        w = w / jnp.sum(w)
        rows.append(jnp.zeros((N_NODES,), jnp.float32).at[t0:t1].set(w))
    rows.append(jnp.zeros((N_NODES,), jnp.float32))
    wrow = jnp.stack(rows)

    tmask_np = np.zeros((N_NODES, N_NODES), np.float32)
    for nid in range(N_NODES - 1):
        tmask_np[nid, nid + 1:min(nid + 1 + FANOUT, N_NODES)] = 1.0
    negb = jnp.asarray((1.0 - tmask_np) * -1e30)
    zrow = jnp.asarray(1.0 - np.eye(N_NODES, dtype=np.float32))
    hot_np = np.zeros((2 * N_NODES, N_NODES), np.float32)
    hot_np[:N_NODES] = np.eye(N_NODES, dtype=np.float32)
    hot = jnp.asarray(hot_np)

    nb = B // BB
    grid = (nb, N_NODES - 1)
    out = pl.pallas_call(
        _stage_kernel,
        grid=grid,
        in_specs=[
            pl.BlockSpec((BB, D_MODEL), lambda b, n: (b, 0)),
            pl.BlockSpec((D_MODEL, EMB_DIMS), lambda b, n: (0, 0)),
            pl.BlockSpec((D_MODEL, EMB_DIMS), lambda b, n: (0, 0)),
            pl.BlockSpec((1, D_MODEL, D_MODEL), lambda b, n: (n, 0, 0)),
            pl.BlockSpec((N_NODES, D_MODEL), lambda b, n: (0, 0)),
            pl.BlockSpec((EMB_DIMS, N_NODES), lambda b, n: (0, 0)),
            pl.BlockSpec((N_NODES, N_NODES), lambda b, n: (0, 0)),
            pl.BlockSpec((N_NODES, N_NODES), lambda b, n: (0, 0)),
            pl.BlockSpec((N_NODES, N_NODES), lambda b, n: (0, 0)),
            pl.BlockSpec((2 * N_NODES, N_NODES), lambda b, n: (0, 0)),
        ],
        out_specs=pl.BlockSpec((BB, D_MODEL), lambda b, n: (b, 0)),
        out_shape=jax.ShapeDtypeStruct((B, D_MODEL), jnp.float32),
        scratch_shapes=[
            pltpu.VMEM((BB, D_MODEL), jnp.float32),
            pltpu.VMEM((BB, D_MODEL), jnp.float32),
            pltpu.VMEM((BB, D_MODEL), jnp.float32),
            pltpu.VMEM((BB, D_MODEL), jnp.float32),
            pltpu.VMEM((BB, N_NODES), jnp.float32),
            pltpu.VMEM((BB, N_NODES), jnp.float32),
            pltpu.VMEM((BB, N_NODES), jnp.float32),
            pltpu.VMEM((BB, N_NODES), jnp.float32),
            pltpu.VMEM((BB, N_NODES), jnp.float32),
        ],
        compiler_params=pltpu.CompilerParams(
            dimension_semantics=("arbitrary", "arbitrary"),
        ),
    )(x, W_attn1, W_attn2, W_trans, b_trans, embsT, wrow, negb, zrow, hot)
    return out
